# baseline (speedup 1.0000x reference)
import functools

import jax
import jax.numpy as jnp
from jax import lax
from jax.experimental import pallas as pl
from jax.experimental.pallas import tpu as pltpu
from jax.experimental.pallas import tpu_sc as plsc


def kernel(vertices, joints, extra_joints_idxs):
    B, V, C = vertices.shape
    J = joints.shape[1]
    K = extra_joints_idxs.shape[0]
    OW = (J + K) * C
    JW = J * C
    GW = K * C
    VC = V * C
    WT = 8

    info = plsc.get_sparse_core_info()
    NC, NS, L = info.num_cores, info.num_subcores, info.num_lanes
    NW = NC * NS
    BPW = B // NW
    NR = BPW * K * 2
    CH = 128
    NJV = -(-JW // L)

    vtab = vertices.reshape(B * VC // WT, WT)
    jflat = joints.reshape(B * JW)
    idx8 = jnp.zeros((8,), jnp.int32).at[:K].set(
        extra_joints_idxs.astype(jnp.int32))

    mesh = plsc.VectorSubcoreMesh(core_axis_name="c", subcore_axis_name="s")

    @functools.partial(
        pl.kernel,
        mesh=mesh,
        out_type=jax.ShapeDtypeStruct((B * OW,), jnp.float32),
        compiler_params=pltpu.CompilerParams(
            use_tc_tiling_on_sc=False, needs_layout_passes=False),
        scratch_types=[
            pltpu.VMEM((8,), jnp.int32),
            pltpu.VMEM((NR,), jnp.int32),
            pltpu.VMEM((NR, WT), jnp.float32),
            pltpu.VMEM((BPW * JW + L,), jnp.float32),
            pltpu.VMEM((BPW * OW + L,), jnp.float32),
            pltpu.SemaphoreType.DMA,
        ],
    )
    def k(vtab_hbm, jflat_hbm, idx_hbm, out_hbm, idx_v, addr_v, g, jbuf,
          obuf, sem):
        wid = lax.axis_index("s") * NC + lax.axis_index("c")
        b0 = wid * BPW

        pltpu.sync_copy(idx_hbm, idx_v)

        def abody(j, _):
            e = lax.iota(jnp.int32, L) + j * L
            t = e // 2
            kk = e - t * 2
            b = t // K
            i = t - b * K
            idxvals = plsc.load_gather(idx_v, [i])
            s = (b0 + b) * VC + idxvals * C
            addr_v[pl.ds(j * L, L)] = s // WT + kk
            return 0

        lax.fori_loop(0, NR // L, abody, 0)

        gathers = []
        for j in range(NR // CH):
            gathers.append(pltpu.async_copy(
                vtab_hbm.at[addr_v.at[pl.ds(j * CH, CH)]],
                g.at[pl.ds(j * CH, CH)], sem))

        pltpu.sync_copy(jflat_hbm.at[pl.ds(b0 * JW, BPW * JW)],
                        jbuf.at[pl.ds(0, BPW * JW)])

        def jbody(b, _):
            for kk in range(NJV):
                src = jbuf[pl.ds(b * JW + kk * L, L)]
                obuf[pl.ds(b * OW + kk * L, L)] = src
            return 0

        lax.fori_loop(0, BPW, jbody, 0)

        for gth in gathers:
            gth.wait()

        wv = lax.iota(jnp.int32, L)
        wrow = wv // C
        wcol = wv - wrow * C
        wrow_c = jnp.minimum(wrow, K - 1)
        idxw = plsc.load_gather(idx_v, [wrow_c])
        gmask = wv < GW

        def gbody(b, _):
            t = jnp.minimum(b * K + wrow_c, BPW * K - 1)
            s = (b0 + b) * VC + idxw * C
            p = t * (2 * WT) + (s - (s // WT) * WT) + wcol
            vals = plsc.load_gather(g, [p // WT, p - (p // WT) * WT])
            plsc.store_compressed(obuf.at[pl.ds(b * OW + JW, L)], vals,
                                  mask=gmask)
            return 0

        lax.fori_loop(0, BPW, gbody, 0)

        pltpu.sync_copy(obuf.at[pl.ds(0, BPW * OW)],
                        out_hbm.at[pl.ds(b0 * OW, BPW * OW)])

    out = k(vtab, jflat, idx8)
    return out.reshape(B, J + K, C)
